# Initial kernel scaffold; baseline (speedup 1.0000x reference)
import jax
import jax.numpy as jnp
from jax.experimental import pallas as pl


def kernel(ids, table):
    raise NotImplementedError("write your pallas kernel here")


# speedup vs baseline: 13.4875x; 13.4875x over previous
import functools
import jax
import jax.numpy as jnp
from jax import lax
from jax.experimental import pallas as pl
from jax.experimental.pallas import tpu as pltpu, tpu_sc as plsc

EMBED = 64
NC = 2
NS = 16
NW = NC * NS

IDX_CHUNK = 128
GATHERS_PER_STEP = 8
STEP_ROWS = IDX_CHUNK * GATHERS_PER_STEP


@functools.partial(jax.jit, static_argnames=("n_steps",))
def _sc_gather(ids2d, table, n_steps):
    rows_per_w = n_steps * STEP_ROWS
    idx_rows_per_w = n_steps * GATHERS_PER_STEP
    total_rows = NW * rows_per_w
    mesh = plsc.VectorSubcoreMesh(core_axis_name="c", subcore_axis_name="s")

    @functools.partial(
        pl.kernel,
        out_type=jax.ShapeDtypeStruct((total_rows, EMBED), jnp.float32),
        mesh=mesh,
        scratch_types=[
            pltpu.VMEM((idx_rows_per_w, IDX_CHUNK), jnp.int32),
            pltpu.VMEM((STEP_ROWS, EMBED), jnp.float32),
            pltpu.SemaphoreType.DMA,
        ],
        compiler_params=pltpu.CompilerParams(use_tc_tiling_on_sc=False),
    )
    def k(ids_hbm, table_hbm, out_hbm, idx_v, rows_v, sem):
        cid = lax.axis_index("c")
        sid = lax.axis_index("s")
        wid = sid * NC + cid
        pltpu.sync_copy(ids_hbm.at[pl.ds(wid * idx_rows_per_w, idx_rows_per_w)],
                        idx_v)
        out_base = wid * rows_per_w

        def step(i, _):
            copies = []
            for j in range(GATHERS_PER_STEP):
                copies.append(pltpu.async_copy(
                    table_hbm.at[idx_v.at[i * GATHERS_PER_STEP + j]],
                    rows_v.at[pl.ds(j * IDX_CHUNK, IDX_CHUNK)],
                    sem,
                ))
            for c in copies:
                c.wait()
            pltpu.sync_copy(rows_v,
                            out_hbm.at[pl.ds(out_base + i * STEP_ROWS,
                                             STEP_ROWS)])
            return 0

        lax.fori_loop(0, n_steps, step, 0)

    return k(ids2d, table)


def kernel(ids, table):
    input_shape = ids.shape
    n = ids.size
    ids_flat = ids.reshape(-1).astype(jnp.int32)
    assert n % (NW * STEP_ROWS) == 0
    n_steps = n // (NW * STEP_ROWS)
    ids2d = ids_flat.reshape(-1, IDX_CHUNK)
    out = _sc_gather(ids2d, table, n_steps)
    return out.reshape(input_shape + (EMBED,))


# speedup vs baseline: 13.5788x; 1.0068x over previous
import functools
import jax
import jax.numpy as jnp
from jax import lax
from jax.experimental import pallas as pl
from jax.experimental.pallas import tpu as pltpu, tpu_sc as plsc

EMBED = 64
NC = 2
NS = 16
NW = NC * NS

IDX_CHUNK = 128
GATHERS_PER_STEP = 4
STEP_ROWS = IDX_CHUNK * GATHERS_PER_STEP
STEP_BYTES = STEP_ROWS * EMBED * 4


@functools.partial(jax.jit, static_argnames=("n_pairs",))
def _sc_gather(ids2d, table, n_pairs):
    n_steps = 2 * n_pairs
    rows_per_w = n_steps * STEP_ROWS
    idx_rows_per_w = n_steps * GATHERS_PER_STEP
    total_rows = NW * rows_per_w
    mesh = plsc.VectorSubcoreMesh(core_axis_name="c", subcore_axis_name="s")

    @functools.partial(
        pl.kernel,
        out_type=jax.ShapeDtypeStruct((total_rows, EMBED), jnp.float32),
        mesh=mesh,
        scratch_types=[
            pltpu.VMEM((idx_rows_per_w, IDX_CHUNK), jnp.int32),
            pltpu.VMEM((2 * STEP_ROWS, EMBED), jnp.float32),
            pltpu.SemaphoreType.DMA,
            pltpu.SemaphoreType.DMA,
            pltpu.SemaphoreType.DMA,
            pltpu.SemaphoreType.DMA,
        ],
        compiler_params=pltpu.CompilerParams(use_tc_tiling_on_sc=False),
    )
    def k(ids_hbm, table_hbm, out_hbm, idx_v, rows_v, sem_g0, sem_g1,
          sem_o0, sem_o1):
        cid = lax.axis_index("c")
        sid = lax.axis_index("s")
        wid = sid * NC + cid
        pltpu.sync_copy(ids_hbm.at[pl.ds(wid * idx_rows_per_w, idx_rows_per_w)],
                        idx_v)
        out_base = wid * rows_per_w
        sem_g = (sem_g0, sem_g1)
        sem_o = (sem_o0, sem_o1)

        def buf(b):
            return rows_v.at[pl.ds(b * STEP_ROWS, STEP_ROWS)]

        def fire_gathers(step, b):
            for j in range(GATHERS_PER_STEP):
                pltpu.async_copy(
                    table_hbm.at[idx_v.at[step * GATHERS_PER_STEP + j]],
                    rows_v.at[pl.ds(b * STEP_ROWS + j * IDX_CHUNK, IDX_CHUNK)],
                    sem_g[b],
                )

        def drain_gathers(b):
            pltpu.make_async_copy(table_hbm.at[pl.ds(0, STEP_ROWS)],
                                  buf(b), sem_g[b]).wait()

        def fire_out(step, b):
            pltpu.async_copy(buf(b),
                             out_hbm.at[pl.ds(out_base + step * STEP_ROWS,
                                              STEP_ROWS)],
                             sem_o[b])

        def drain_out(b):
            pltpu.make_async_copy(table_hbm.at[pl.ds(0, STEP_ROWS)],
                                  buf(b), sem_o[b]).wait()

        fire_gathers(0, 0)

        def pair(p, _):
            s0 = 2 * p
            drain_gathers(0)
            fire_out(s0, 0)

            @pl.when(p > 0)
            def _():
                drain_out(1)
            fire_gathers(s0 + 1, 1)
            drain_gathers(1)
            fire_out(s0 + 1, 1)
            drain_out(0)

            @pl.when(p + 1 < n_pairs)
            def _():
                fire_gathers(s0 + 2, 0)
            return 0

        lax.fori_loop(0, n_pairs, pair, 0)
        drain_out(1)

    return k(ids2d, table)


def kernel(ids, table):
    input_shape = ids.shape
    n = ids.size
    ids_flat = ids.reshape(-1).astype(jnp.int32)
    assert n % (NW * 2 * STEP_ROWS) == 0
    n_pairs = n // (NW * 2 * STEP_ROWS)
    ids2d = ids_flat.reshape(-1, IDX_CHUNK)
    out = _sc_gather(ids2d, table, n_pairs)
    return out.reshape(input_shape + (EMBED,))
